# Initial kernel scaffold; baseline (speedup 1.0000x reference)
import jax
import jax.numpy as jnp
from jax.experimental import pallas as pl


def kernel(x, xlens, y):
    raise NotImplementedError("write your pallas kernel here")


# speedup vs baseline: 10.2525x; 10.2525x over previous
import functools

import jax
import jax.numpy as jnp
from jax import lax
from jax.experimental import pallas as pl
from jax.experimental.pallas import tpu as pltpu
from jax.experimental.pallas import tpu_sc as plsc

LOGZERO = -1.0e10
B, T, O = 8, 256, 1000
H = 4
NBH = B * H
W = 256
L = 16
NCHUNK = W // L
LN2 = 0.6931471805599453


def _vlog(s):
    i = lax.bitcast_convert_type(s, jnp.int32)
    e = lax.shift_right_arithmetic(i, 23) - 127
    m = lax.bitcast_convert_type(
        jnp.bitwise_or(jnp.bitwise_and(i, jnp.int32(0x7FFFFF)),
                       jnp.int32(127 << 23)), jnp.float32)
    z = (m - 1.0) / (m + 1.0)
    z2 = z * z
    p = 2.0 * z * (1.0 + z2 * (1.0 / 3.0 + z2 * (1.0 / 5.0
                   + z2 * (1.0 / 7.0 + z2 * (1.0 / 9.0)))))
    return e.astype(jnp.float32) * LN2 + p


def _splat_i32(v):
    return jnp.full((L,), v, dtype=jnp.int32)


_mesh = plsc.VectorSubcoreMesh(core_axis_name="c", subcore_axis_name="s")


@functools.partial(
    pl.kernel, mesh=_mesh,
    out_type=jax.ShapeDtypeStruct((NBH, O), jnp.float32),
    compiler_params=pltpu.CompilerParams(use_tc_tiling_on_sc=False,
                                         needs_layout_passes=False),
    scratch_types=[
        pltpu.VMEM((T, W), jnp.float32),
        pltpu.VMEM((T, L), jnp.float32),
        pltpu.VMEM((T,), jnp.float32),
        pltpu.VMEM((B,), jnp.int32),
        pltpu.VMEM((W,), jnp.float32),
    ],
)
def _ctc_kernel(x_hbm, xlens_hbm, y_hbm, out_hbm, xv, xb, cv, xlv, ov):
    del y_hbm
    wid = lax.axis_index("c") * 16 + lax.axis_index("s")
    b = wid // 4
    k = wid % 4
    o0 = k * W - (k // 3) * 24

    pltpu.sync_copy(xlens_hbm, xlv)
    pltpu.sync_copy(x_hbm.at[b, :, pl.ds(o0, W)], xv)
    pltpu.sync_copy(x_hbm.at[b, :, pl.ds(0, L)], xb)

    lanes = lax.iota(jnp.int32, L)
    xlen = jnp.max(plsc.load_gather(xlv, [_splat_i32(b)]))

    zeros_i = _splat_i32(0)
    carry = jnp.float32(0.0)
    for kc in range(T // L):
        blk = plsc.load_gather(xb, [lanes + (kc * L), zeros_i])
        cv[pl.ds(kc * L, L)] = plsc.cumsum(blk) + carry
        carry = carry + jnp.sum(blk)

    zgate = jnp.where(xlen == T, jnp.float32(0.0), jnp.float32(LOGZERO))
    cend = plsc.load_gather(cv, [_splat_i32(xlen - 1)])

    for j in range(NCHUNK):
        co = j * L
        a0 = xv[0, pl.ds(co, L)]

        def p1_body(i, mp, co=co):
            mv, pv = mp
            t = xlen - i
            cvec = plsc.load_gather(cv, [_splat_i32(t - 1)])
            a = xv[t, pl.ds(co, L)]
            return jnp.maximum(mv, cvec + a), pv + a

        m0 = jnp.full((L,), LOGZERO, dtype=jnp.float32)
        p0 = jnp.zeros((L,), dtype=jnp.float32)
        mv, pv = lax.fori_loop(1, xlen, p1_body, (m0, p0))
        term0 = pv + a0 + zgate
        mv = jnp.maximum(mv, term0)
        mz = mv - zgate

        def p2_body(i, sp, co=co, mv=mv, mz=mz):
            sv, pv2 = sp
            t = xlen - i
            cvec = plsc.load_gather(cv, [_splat_i32(t - 1)])
            a = xv[t, pl.ds(co, L)]
            pv2 = pv2 + a
            e1 = jnp.exp(cvec + a - mv)
            e2 = jnp.exp(cvec + pv2 - mz)
            return sv + e1 + e2, pv2

        sv, _ = lax.fori_loop(1, xlen, p2_body,
                              (jnp.zeros((L,), jnp.float32), p0))
        sv = sv + jnp.exp(term0 - mv)

        res = mv + _vlog(sv)
        oabs = lanes + (o0 + co)
        res = jnp.where(oabs == (O - 1), cend, res)
        res = jnp.where(oabs == 0, jnp.float32(LOGZERO), res)
        ov[pl.ds(co, L)] = res

    for h in range(H):
        pltpu.sync_copy(ov, out_hbm.at[b * H + h, pl.ds(o0, W)])


def kernel(x, xlens, y):
    return _ctc_kernel(x, xlens.astype(jnp.int32), y)


# speedup vs baseline: 16.1313x; 1.5734x over previous
import functools

import jax
import jax.numpy as jnp
from jax import lax
from jax.experimental import pallas as pl
from jax.experimental.pallas import tpu as pltpu
from jax.experimental.pallas import tpu_sc as plsc

LOGZERO = -1.0e10
B, T, O = 8, 256, 1000
H = 4
NBH = B * H
W = 256
OP = 4 * W
L = 16
NCHUNK = W // L
LN2 = 0.6931471805599453


def _vlog(s):
    i = lax.bitcast_convert_type(s, jnp.int32)
    e = lax.shift_right_arithmetic(i, 23) - 127
    m = lax.bitcast_convert_type(
        jnp.bitwise_or(jnp.bitwise_and(i, jnp.int32(0x7FFFFF)),
                       jnp.int32(127 << 23)), jnp.float32)
    z = (m - 1.0) / (m + 1.0)
    z2 = z * z
    p = 2.0 * z * (1.0 + z2 * (1.0 / 3.0 + z2 * (1.0 / 5.0
                   + z2 * (1.0 / 7.0 + z2 * (1.0 / 9.0)))))
    return e.astype(jnp.float32) * LN2 + p


def _splat_i32(v):
    return jnp.full((L,), v, dtype=jnp.int32)


_mesh = plsc.VectorSubcoreMesh(core_axis_name="c", subcore_axis_name="s")


@functools.partial(
    pl.kernel, mesh=_mesh,
    out_type=jax.ShapeDtypeStruct((B, H, OP), jnp.float32),
    compiler_params=pltpu.CompilerParams(needs_layout_passes=False),
    scratch_types=[
        pltpu.VMEM((T, W), jnp.float32),
        pltpu.VMEM((T, 128), jnp.float32),
        pltpu.VMEM((T,), jnp.float32),
        pltpu.VMEM((B,), jnp.int32),
        pltpu.VMEM((H, W), jnp.float32),
    ],
)
def _ctc_kernel(x_hbm, xlens_hbm, y_hbm, out_hbm, xv, xb, cv, xlv, ov):
    del y_hbm
    wid = lax.axis_index("c") * 16 + lax.axis_index("s")
    b = wid // 4
    k = wid % 4
    o0 = k * W

    pltpu.sync_copy(xlens_hbm, xlv)
    pltpu.sync_copy(x_hbm.at[b, :, pl.ds(o0, W)], xv)
    pltpu.sync_copy(x_hbm.at[b, :, pl.ds(0, 128)], xb)

    lanes = lax.iota(jnp.int32, L)
    xlen = jnp.max(plsc.load_gather(xlv, [_splat_i32(b)]))

    zeros_i = _splat_i32(0)
    carry = jnp.float32(0.0)
    for kc in range(T // L):
        blk = plsc.load_gather(xb, [lanes + (kc * L), zeros_i])
        cv[pl.ds(kc * L, L)] = plsc.cumsum(blk) + carry
        carry = carry + jnp.sum(blk)

    zgate = jnp.where(xlen == T, jnp.float32(0.0), jnp.float32(LOGZERO))
    cend = plsc.load_gather(cv, [_splat_i32(xlen - 1)])
    fzero = jnp.zeros((L,), dtype=jnp.float32)
    minit = jnp.full((L,), LOGZERO, dtype=jnp.float32)

    G1 = 8
    mv = []
    for g in range(NCHUNK // G1):
        def p1_body(i, ms, g=g):
            cvec = plsc.load_gather(cv, [_splat_i32(i - 1)])
            return tuple(
                jnp.maximum(ms[u], cvec + xv[i, pl.ds((g * G1 + u) * L, L)])
                for u in range(G1))
        mv.extend(lax.fori_loop(1, xlen, p1_body, (minit,) * G1))

    sv = []
    for g in range(NCHUNK // G1):
        def p2_body(i, ss, g=g):
            cvec = plsc.load_gather(cv, [_splat_i32(i - 1)])
            return tuple(
                ss[u] + jnp.exp(cvec + xv[i, pl.ds((g * G1 + u) * L, L)]
                                - mv[g * G1 + u])
                for u in range(G1))
        sv.extend(lax.fori_loop(1, xlen, p2_body, (fzero,) * G1))
    sv = list(sv)

    G3 = 4
    ptot = []
    for g in range(NCHUNK // G3):
        def sfx_loop():
            def sfx_body(i, carry, g=g):
                t = T - 1 - i
                cvec = plsc.load_gather(cv, [_splat_i32(t - 1)])
                out = []
                for u in range(G3):
                    s, p = carry[2 * u], carry[2 * u + 1]
                    p = p + xv[t, pl.ds((g * G3 + u) * L, L)]
                    s = s + jnp.exp(cvec + p - mv[g * G3 + u])
                    out.extend((s, p))
                return tuple(out)
            return lax.fori_loop(0, T - 1, sfx_body, (fzero,) * (2 * G3))

        def sfx_skip():
            return (fzero,) * (2 * G3)

        res = lax.cond(xlen == T, sfx_loop, sfx_skip)
        for u in range(G3):
            sv[g * G3 + u] = sv[g * G3 + u] + res[2 * u]
            ptot.append(res[2 * u + 1])

    for j in range(NCHUNK):
        a0 = xv[0, pl.ds(j * L, L)]
        term0 = ptot[j] + a0 + zgate
        m2 = jnp.maximum(mv[j], term0)
        s2 = sv[j] * jnp.exp(mv[j] - m2) + jnp.exp(term0 - m2)
        res = m2 + _vlog(s2)
        oabs = lanes + (o0 + j * L)
        res = jnp.where(oabs == (O - 1), cend, res)
        res = jnp.where(oabs == 0, jnp.float32(LOGZERO), res)
        for h in range(H):
            ov[h, pl.ds(j * L, L)] = res

    pltpu.sync_copy(ov, out_hbm.at[b, :, pl.ds(o0, W)])


def kernel(x, xlens, y):
    xp = jnp.concatenate(
        [x, jnp.full((B, T, OP - O), LOGZERO, dtype=x.dtype)], axis=2)
    out = _ctc_kernel(xp, xlens.astype(jnp.int32), y)
    return out.reshape(NBH, OP)[:, :O]


# speedup vs baseline: 16.1874x; 1.0035x over previous
import functools

import jax
import jax.numpy as jnp
from jax import lax
from jax.experimental import pallas as pl
from jax.experimental.pallas import tpu as pltpu
from jax.experimental.pallas import tpu_sc as plsc

LOGZERO = -1.0e10
B, T, O = 8, 256, 1000
H = 4
NBH = B * H
W = 256
L = 16
NCHUNK = W // L
TH = T // 2
LN2 = 0.6931471805599453


def _vlog(s):
    i = lax.bitcast_convert_type(s, jnp.int32)
    e = lax.shift_right_arithmetic(i, 23) - 127
    m = lax.bitcast_convert_type(
        jnp.bitwise_or(jnp.bitwise_and(i, jnp.int32(0x7FFFFF)),
                       jnp.int32(127 << 23)), jnp.float32)
    z = (m - 1.0) / (m + 1.0)
    z2 = z * z
    p = 2.0 * z * (1.0 + z2 * (1.0 / 3.0 + z2 * (1.0 / 5.0
                   + z2 * (1.0 / 7.0 + z2 * (1.0 / 9.0)))))
    return e.astype(jnp.float32) * LN2 + p


def _splat_i32(v):
    return jnp.full((L,), v, dtype=jnp.int32)


_mesh = plsc.VectorSubcoreMesh(core_axis_name="c", subcore_axis_name="s")


@functools.partial(
    pl.kernel, mesh=_mesh,
    out_type=jax.ShapeDtypeStruct((NBH, O), jnp.float32),
    compiler_params=pltpu.CompilerParams(use_tc_tiling_on_sc=False,
                                         needs_layout_passes=False),
    scratch_types=[
        pltpu.VMEM((T, W), jnp.float32),
        pltpu.VMEM((T, L), jnp.float32),
        pltpu.VMEM((T,), jnp.float32),
        pltpu.VMEM((B,), jnp.int32),
        pltpu.VMEM((H, W), jnp.float32),
        pltpu.SemaphoreType.DMA,
        pltpu.SemaphoreType.DMA,
        pltpu.SemaphoreType.DMA,
    ],
)
def _ctc_kernel(x_hbm, xlens_hbm, y_hbm, out_hbm, xv, xb, cv, xlv, ov,
                semb, sem1, sem2):
    del y_hbm
    wid = lax.axis_index("c") * 16 + lax.axis_index("s")
    b = wid // 4
    k = wid % 4
    o0 = k * W - (k // 3) * 24

    cpb = pltpu.async_copy(x_hbm.at[b, :, pl.ds(0, L)], xb, semb)
    cp1 = pltpu.async_copy(x_hbm.at[b, pl.ds(0, TH), pl.ds(o0, W)],
                           xv.at[pl.ds(0, TH)], sem1)
    cp2 = pltpu.async_copy(x_hbm.at[b, pl.ds(TH, T - TH), pl.ds(o0, W)],
                           xv.at[pl.ds(TH, T - TH)], sem2)
    pltpu.sync_copy(xlens_hbm, xlv)

    lanes = lax.iota(jnp.int32, L)
    xlen = jnp.max(plsc.load_gather(xlv, [_splat_i32(b)]))

    cpb.wait()
    zeros_i = _splat_i32(0)
    carry = jnp.float32(0.0)
    for kc in range(T // L):
        blk = plsc.load_gather(xb, [lanes + (kc * L), zeros_i])
        cv[pl.ds(kc * L, L)] = plsc.cumsum(blk) + carry
        carry = carry + jnp.sum(blk)

    zgate = jnp.where(xlen == T, jnp.float32(0.0), jnp.float32(LOGZERO))
    cend = plsc.load_gather(cv, [_splat_i32(xlen - 1)])
    fzero = jnp.zeros((L,), dtype=jnp.float32)
    minit = jnp.full((L,), LOGZERO, dtype=jnp.float32)
    xlen_lo = jnp.minimum(xlen, TH)

    G1 = 8
    def p1_group(g, lo, hi, ms):
        def p1_body(i, ms, g=g):
            cvec = plsc.load_gather(cv, [_splat_i32(i - 1)])
            return tuple(
                jnp.maximum(ms[u], cvec + xv[i, pl.ds((g * G1 + u) * L, L)])
                for u in range(G1))
        return lax.fori_loop(lo, hi, p1_body, ms)

    cp1.wait()
    mlo = [p1_group(g, 1, xlen_lo, (minit,) * G1) for g in range(NCHUNK // G1)]
    cp2.wait()
    mv = []
    for g in range(NCHUNK // G1):
        mv.extend(p1_group(g, xlen_lo, xlen, mlo[g]))

    sv = []
    for g in range(NCHUNK // G1):
        def p2_body(i, ss, g=g):
            cvec = plsc.load_gather(cv, [_splat_i32(i - 1)])
            return tuple(
                ss[u] + jnp.exp(cvec + xv[i, pl.ds((g * G1 + u) * L, L)]
                                - mv[g * G1 + u])
                for u in range(G1))
        sv.extend(lax.fori_loop(1, xlen, p2_body, (fzero,) * G1))
    sv = list(sv)

    G3 = 4
    ptot = []
    for g in range(NCHUNK // G3):
        def sfx_loop(g=g):
            def sfx_body(i, carry, g=g):
                t = T - 1 - i
                cvec = plsc.load_gather(cv, [_splat_i32(t - 1)])
                out = []
                for u in range(G3):
                    s, p = carry[2 * u], carry[2 * u + 1]
                    p = p + xv[t, pl.ds((g * G3 + u) * L, L)]
                    s = s + jnp.exp(cvec + p - mv[g * G3 + u])
                    out.extend((s, p))
                return tuple(out)
            return lax.fori_loop(0, T - 1, sfx_body, (fzero,) * (2 * G3))

        def sfx_skip():
            return (fzero,) * (2 * G3)

        res = lax.cond(xlen == T, sfx_loop, sfx_skip)
        for u in range(G3):
            sv[g * G3 + u] = sv[g * G3 + u] + res[2 * u]
            ptot.append(res[2 * u + 1])

    for j in range(NCHUNK):
        a0 = xv[0, pl.ds(j * L, L)]
        term0 = ptot[j] + a0 + zgate
        m2 = jnp.maximum(mv[j], term0)
        s2 = sv[j] * jnp.exp(mv[j] - m2) + jnp.exp(term0 - m2)
        res = m2 + _vlog(s2)
        oabs = lanes + (o0 + j * L)
        res = jnp.where(oabs == (O - 1), cend, res)
        res = jnp.where(oabs == 0, jnp.float32(LOGZERO), res)
        for h in range(H):
            ov[h, pl.ds(j * L, L)] = res

    pltpu.sync_copy(ov, out_hbm.at[pl.ds(b * H, H), pl.ds(o0, W)])


def kernel(x, xlens, y):
    return _ctc_kernel(x, xlens.astype(jnp.int32), y)
